# baseline (speedup 1.0000x reference)
import functools

import jax
import jax.numpy as jnp
from jax import lax
from jax.experimental import pallas as pl
from jax.experimental.pallas import tpu as pltpu
from jax.experimental.pallas import tpu_sc as plsc

BUCKETS = 50
D = 2048
N_ROWS = 16384
LANES = 16
BLOCK_N = 1024
SC_CORES = 1

_info = plsc.get_sparse_core_info()
NC, NS = SC_CORES, _info.num_subcores
NW = NC * NS
ROWS_PER_TILE = N_ROWS // NW


def _sc_bucket_idx(labels_hbm, idx_hbm, labv, idxv):
    wid = lax.axis_index("s") * NC + lax.axis_index("c")
    base = wid * ROWS_PER_TILE
    pltpu.sync_copy(labels_hbm.at[pl.ds(base, ROWS_PER_TILE)], labv)

    def idx_body(j, c):
        lab = labv[pl.ds(j * LANES, LANES)]
        idxv[pl.ds(j * LANES, LANES)] = jnp.where(
            lab <= 1.0, jnp.int32(BUCKETS - 1), jnp.int32(0))
        return c
    lax.fori_loop(0, ROWS_PER_TILE // LANES, idx_body, 0)
    pltpu.sync_copy(idxv, idx_hbm.at[pl.ds(base, ROWS_PER_TILE)])


def _tc_main(idx_ref, features_ref, m1_ref, v1_ref, m2_ref, v2_ref,
             out_ref, scale_ref, bias_ref):
    @pl.when(pl.program_id(0) == 0)
    def _prep():
        scale = jnp.sqrt(jnp.clip(v2_ref[...] / v1_ref[...], 0.5, 2.0))
        scale_ref[...] = scale
        bias_ref[...] = m2_ref[...] - m1_ref[...] * scale

    hi = idx_ref[...] == BUCKETS - 1
    row_scale = jnp.where(hi, scale_ref[BUCKETS - 1, :][None, :],
                          scale_ref[0, :][None, :])
    row_bias = jnp.where(hi, bias_ref[BUCKETS - 1, :][None, :],
                         bias_ref[0, :][None, :])
    out_ref[...] = features_ref[...] * row_scale + row_bias


@functools.partial(jax.jit, static_argnames=())
def kernel(features, labels, epoch, running_mean_last_epoch,
           running_var_last_epoch, smoothed_mean_last_epoch,
           smoothed_var_last_epoch):
    n = features.shape[0]
    grid = n // BLOCK_N
    smooth = epoch >= 1
    m1 = jnp.where(smooth, running_mean_last_epoch, 0.0)
    v1 = jnp.where(smooth, running_var_last_epoch, 1.0)
    m2 = jnp.where(smooth, smoothed_mean_last_epoch, 0.0)
    v2 = jnp.where(smooth, smoothed_var_last_epoch, 1.0)

    mesh = plsc.VectorSubcoreMesh(core_axis_name="c", subcore_axis_name="s",
                                  num_cores=SC_CORES)
    idx = functools.partial(
        pl.kernel, mesh=mesh,
        out_type=jax.ShapeDtypeStruct((n,), jnp.int32),
        scratch_types=[
            pltpu.VMEM((ROWS_PER_TILE,), jnp.float32),
            pltpu.VMEM((ROWS_PER_TILE,), jnp.int32),
        ],
    )(_sc_bucket_idx)(labels)
    idx2 = idx.reshape(n, 1)

    table_spec = pl.BlockSpec((BUCKETS, D), lambda i: (0, 0))
    return pl.pallas_call(
        _tc_main,
        grid=(grid,),
        in_specs=[
            pl.BlockSpec((BLOCK_N, 1), lambda i: (i, 0)),
            pl.BlockSpec((BLOCK_N, D), lambda i: (i, 0)),
            table_spec, table_spec, table_spec, table_spec,
        ],
        out_specs=pl.BlockSpec((BLOCK_N, D), lambda i: (i, 0)),
        out_shape=jax.ShapeDtypeStruct((n, D), jnp.float32),
        scratch_shapes=[
            pltpu.VMEM((BUCKETS, D), jnp.float32),
            pltpu.VMEM((BUCKETS, D), jnp.float32),
        ],
    )(idx2, features, m1, v1, m2, v2)


# speedup vs baseline: 1.0916x; 1.0916x over previous
import functools

import jax
import jax.numpy as jnp
from jax import lax
from jax.experimental import pallas as pl
from jax.experimental.pallas import tpu as pltpu
from jax.experimental.pallas import tpu_sc as plsc

BUCKETS = 50
D = 2048
N_ROWS = 16384
LANES = 16
BLOCK_N = 1024
SC_CORES = 1

_info = plsc.get_sparse_core_info()
NC, NS = SC_CORES, _info.num_subcores
NW = NC * NS
ROWS_PER_TILE = N_ROWS // NW


def _sc_bucket_idx(labels_hbm, idx_hbm, labv, idxv):
    wid = lax.axis_index("s") * NC + lax.axis_index("c")
    base = wid * ROWS_PER_TILE
    pltpu.sync_copy(labels_hbm.at[pl.ds(base, ROWS_PER_TILE)], labv)

    def idx_body(j, c):
        lab = labv[pl.ds(j * LANES, LANES)]
        idxv[pl.ds(j * LANES, LANES)] = jnp.where(
            lab <= 1.0, jnp.int32(BUCKETS - 1), jnp.int32(0))
        return c
    lax.fori_loop(0, ROWS_PER_TILE // LANES, idx_body, 0)
    pltpu.sync_copy(idxv, idx_hbm.at[pl.ds(base, ROWS_PER_TILE)])


def _tc_main(idx_ref, features_ref, m1_ref, v1_ref, m2_ref, v2_ref,
             out_ref, scale_ref, bias_ref):
    @pl.when(pl.program_id(0) == 0)
    def _prep():
        scale = jnp.sqrt(jnp.clip(v2_ref[...] / v1_ref[...], 0.5, 2.0))
        scale_ref[...] = scale
        bias_ref[...] = m2_ref[...] - m1_ref[...] * scale

    idx = idx_ref[0, 0, :]
    onehot = (idx[:, None] == lax.broadcasted_iota(jnp.int32, (1, BUCKETS), 1)
              ).astype(jnp.float32)
    row_scale = jnp.dot(onehot, scale_ref[...],
                        preferred_element_type=jnp.float32)
    row_bias = jnp.dot(onehot, bias_ref[...],
                       preferred_element_type=jnp.float32)
    out_ref[...] = features_ref[...] * row_scale + row_bias


@functools.partial(jax.jit, static_argnames=())
def kernel(features, labels, epoch, running_mean_last_epoch,
           running_var_last_epoch, smoothed_mean_last_epoch,
           smoothed_var_last_epoch):
    n = features.shape[0]
    grid = n // BLOCK_N
    smooth = epoch >= 1
    m1 = jnp.where(smooth, running_mean_last_epoch, 0.0)
    v1 = jnp.where(smooth, running_var_last_epoch, 1.0)
    m2 = jnp.where(smooth, smoothed_mean_last_epoch, 0.0)
    v2 = jnp.where(smooth, smoothed_var_last_epoch, 1.0)

    mesh = plsc.VectorSubcoreMesh(core_axis_name="c", subcore_axis_name="s",
                                  num_cores=SC_CORES)
    idx = functools.partial(
        pl.kernel, mesh=mesh,
        out_type=jax.ShapeDtypeStruct((n,), jnp.int32),
        scratch_types=[
            pltpu.VMEM((ROWS_PER_TILE,), jnp.float32),
            pltpu.VMEM((ROWS_PER_TILE,), jnp.int32),
        ],
    )(_sc_bucket_idx)(labels)
    idx3 = idx.reshape(grid, 1, BLOCK_N)

    table_spec = pl.BlockSpec((BUCKETS, D), lambda i: (0, 0))
    return pl.pallas_call(
        _tc_main,
        grid=(grid,),
        in_specs=[
            pl.BlockSpec((1, 1, BLOCK_N), lambda i: (i, 0, 0)),
            pl.BlockSpec((BLOCK_N, D), lambda i: (i, 0)),
            table_spec, table_spec, table_spec, table_spec,
        ],
        out_specs=pl.BlockSpec((BLOCK_N, D), lambda i: (i, 0)),
        out_shape=jax.ShapeDtypeStruct((n, D), jnp.float32),
        scratch_shapes=[
            pltpu.VMEM((BUCKETS, D), jnp.float32),
            pltpu.VMEM((BUCKETS, D), jnp.float32),
        ],
    )(idx3, features, m1, v1, m2, v2)
